# Initial kernel scaffold; baseline (speedup 1.0000x reference)
import jax
import jax.numpy as jnp
from jax.experimental import pallas as pl


def kernel(x, edge_index, y, Wl1, bl1, Wr1, Wl2, bl2, Wr2, Wo1, bo1, Wo2, bo2, Wo3, bo3, Wo4, bo4, Wf1, bf1, Wf2, bf2, Wf3, bf3):
    raise NotImplementedError("write your pallas kernel here")


# speedup vs baseline: 3.3914x; 3.3914x over previous
import functools

import jax
import jax.numpy as jnp
from jax import lax
from jax.experimental import pallas as pl
from jax.experimental.pallas import tpu as pltpu
from jax.experimental.pallas import tpu_sc as plsc

NC = 2
NS = 16
CHUNK = 80


def _mesh():
    return plsc.VectorSubcoreMesh(
        core_axis_name="c", subcore_axis_name="s", num_cores=NC, num_subcores=NS
    )


def _row_split(s, n_rows, fn):
    nps = (-(-n_rows // NS) + 7) // 8 * 8
    last = n_rows - (NS - 1) * nps

    @pl.when(s < NS - 1)
    def _():
        fn(s * nps, nps)

    @pl.when(s == NS - 1)
    def _():
        fn((NS - 1) * nps, last)


def _sc_l1(src, dst, x, zeros16, ones_rows, N, E):
    EW = E // (NC * NS)

    def body(src_hbm, dst_hbm, x_hbm, z_hbm, ones_hbm,
             xsum_out, deg_out,
             acc_x, acc_d, idx_s, idx_d, rows, ones_v, sem):
        c = lax.axis_index("c")
        s = lax.axis_index("s")
        w = c * NS + s

        def zero(r0, nr):
            pltpu.sync_copy(z_hbm.at[pl.ds(r0, nr)], acc_x.at[pl.ds(r0, nr)])
            pltpu.sync_copy(z_hbm.at[pl.ds(r0, nr)], acc_d.at[pl.ds(r0, nr)])

        _row_split(s, N, zero)
        pltpu.sync_copy(ones_hbm, ones_v)
        plsc.subcore_barrier()

        base_w = w * EW

        def chunk(j, carry):
            base = base_w + j * CHUNK
            pltpu.sync_copy(src_hbm.at[pl.ds(base, CHUNK)], idx_s)
            pltpu.sync_copy(dst_hbm.at[pl.ds(base, CHUNK)], idx_d)
            pltpu.async_copy(x_hbm.at[idx_s], rows, sem).wait()
            pltpu.sync_copy(rows, acc_x.at[idx_d], add=True)
            pltpu.sync_copy(ones_v, acc_d.at[idx_d], add=True)
            return carry

        lax.fori_loop(0, EW // CHUNK, chunk, 0)
        plsc.subcore_barrier()

        def writeout(r0, nr):
            pltpu.sync_copy(acc_x.at[pl.ds(r0, nr)],
                            xsum_out.at[pl.ds(c * N + r0, nr)])
            pltpu.sync_copy(acc_d.at[pl.ds(r0, nr)],
                            deg_out.at[pl.ds(c * N + r0, nr)])

        _row_split(s, N, writeout)

    f = pl.kernel(
        body,
        out_type=(
            jax.ShapeDtypeStruct((NC * N, 16), jnp.float32),
            jax.ShapeDtypeStruct((NC * N, 16), jnp.float32),
        ),
        mesh=_mesh(),
        compiler_params=pltpu.CompilerParams(use_tc_tiling_on_sc=False),
        scratch_types=[
            pltpu.VMEM_SHARED((N, 16), jnp.float32),
            pltpu.VMEM_SHARED((N, 16), jnp.float32),
            pltpu.VMEM((CHUNK,), jnp.int32),
            pltpu.VMEM((CHUNK,), jnp.int32),
            pltpu.VMEM((CHUNK, 16), jnp.float32),
            pltpu.VMEM((CHUNK, 16), jnp.float32),
            pltpu.SemaphoreType.DMA,
        ],
    )
    return f(src, dst, x, zeros16, ones_rows)


def _sc_l2(src, dst, h1s, zeros32, N, E):
    EW = E // NS
    NBLK = CHUNK // 16

    def body(src_hbm, dst_hbm, t_hbm, z_hbm, out_hbm,
             acc, idx_s, idx_d, rows, sem):
        c = lax.axis_index("c")
        s = lax.axis_index("s")
        _row_split(s, N, lambda r0, nr: pltpu.sync_copy(
            z_hbm.at[pl.ds(r0, nr)], acc.at[pl.ds(r0, nr)]))
        plsc.subcore_barrier()

        base_w = s * EW
        off = c * N

        def chunk(j, carry):
            base = base_w + j * CHUNK
            pltpu.sync_copy(src_hbm.at[pl.ds(base, CHUNK)], idx_s)
            pltpu.sync_copy(dst_hbm.at[pl.ds(base, CHUNK)], idx_d)
            for k in range(NBLK):
                sl = pl.ds(k * 16, 16)
                idx_s[sl] = idx_s[sl] + off
            pltpu.async_copy(t_hbm.at[idx_s], rows, sem).wait()
            pltpu.sync_copy(rows, acc.at[idx_d], add=True)
            return carry

        lax.fori_loop(0, EW // CHUNK, chunk, 0)
        plsc.subcore_barrier()
        _row_split(s, N, lambda r0, nr: pltpu.sync_copy(
            acc.at[pl.ds(r0, nr)], out_hbm.at[pl.ds(c * N + r0, nr)]))

    f = pl.kernel(
        body,
        out_type=jax.ShapeDtypeStruct((NC * N, 32), jnp.float32),
        mesh=_mesh(),
        compiler_params=pltpu.CompilerParams(use_tc_tiling_on_sc=False),
        scratch_types=[
            pltpu.VMEM_SHARED((N, 32), jnp.float32),
            pltpu.VMEM((CHUNK,), jnp.int32),
            pltpu.VMEM((CHUNK,), jnp.int32),
            pltpu.VMEM((CHUNK, 32), jnp.float32),
            pltpu.SemaphoreType.DMA,
        ],
    )
    return f(src, dst, h1s, zeros32)


def _sc_pair_gather(src, dst, tab, N, E):
    EW = E // (NC * NS)
    NBLK = CHUNK // 16

    def body(src_hbm, dst_hbm, t_hbm, ga_out, gb_out,
             idx_s, idx_d, buf_a, buf_b, sem):
        c = lax.axis_index("c")
        s = lax.axis_index("s")
        w = c * NS + s
        base_w = w * EW

        def chunk(j, carry):
            base = base_w + j * CHUNK
            pltpu.sync_copy(src_hbm.at[pl.ds(base, CHUNK)], idx_s)
            pltpu.sync_copy(dst_hbm.at[pl.ds(base, CHUNK)], idx_d)
            for k in range(NBLK):
                sl = pl.ds(k * 16, 16)
                idx_d[sl] = idx_d[sl] + N
            pltpu.async_copy(t_hbm.at[idx_s], buf_a, sem).wait()
            pltpu.sync_copy(buf_a, ga_out.at[pl.ds(base, CHUNK)])
            pltpu.async_copy(t_hbm.at[idx_d], buf_b, sem).wait()
            pltpu.sync_copy(buf_b, gb_out.at[pl.ds(base, CHUNK)])
            return carry

        lax.fori_loop(0, EW // CHUNK, chunk, 0)

    f = pl.kernel(
        body,
        out_type=(
            jax.ShapeDtypeStruct((E, 128), jnp.float32),
            jax.ShapeDtypeStruct((E, 128), jnp.float32),
        ),
        mesh=_mesh(),
        compiler_params=pltpu.CompilerParams(use_tc_tiling_on_sc=False),
        scratch_types=[
            pltpu.VMEM((CHUNK,), jnp.int32),
            pltpu.VMEM((CHUNK,), jnp.int32),
            pltpu.VMEM((CHUNK, 128), jnp.float32),
            pltpu.VMEM((CHUNK, 128), jnp.float32),
            pltpu.SemaphoreType.DMA,
        ],
    )
    return f(src, dst, tab)


def _tc1_body(px_ref, pd_ref, x_ref, wl_ref, bl_ref, wr_ref, out_ref):
    px = px_ref[...]
    pd = pd_ref[...]
    xs = px[0] + px[1]
    deg = pd[0, :, 0:1] + pd[1, :, 0:1]
    inv = 1.0 / jnp.maximum(deg, 1.0)
    agg = xs * inv
    h = jnp.dot(agg, wl_ref[...], preferred_element_type=jnp.float32)
    h = h + bl_ref[...]
    h = h + jnp.dot(x_ref[...], wr_ref[...], preferred_element_type=jnp.float32)
    h = jnp.maximum(h, 0.0)
    out_ref[0] = h[:, :32]
    out_ref[1] = h[:, 32:]


def _tc2_body(a2_ref, pd_ref, h1s_ref, wl_ref, bl_ref, wr_ref, wo1_ref,
              bo1_ref, out_ref):
    a2 = a2_ref[...]
    agg2 = jnp.concatenate([a2[0], a2[1]], axis=1)
    pd = pd_ref[...]
    deg = pd[0, :, 0:1] + pd[1, :, 0:1]
    inv = 1.0 / jnp.maximum(deg, 1.0)
    h1s = h1s_ref[...]
    h1 = jnp.concatenate([h1s[0], h1s[1]], axis=1)
    h2 = jnp.dot(agg2 * inv, wl_ref[...], preferred_element_type=jnp.float32)
    h2 = h2 + bl_ref[...]
    h2 = h2 + jnp.dot(h1, wr_ref[...], preferred_element_type=jnp.float32)
    h2 = jnp.maximum(h2, 0.0)
    wo1 = wo1_ref[...]
    out_ref[0] = jnp.dot(h2, wo1[:128], preferred_element_type=jnp.float32) + bo1_ref[...]
    out_ref[1] = jnp.dot(h2, wo1[128:], preferred_element_type=jnp.float32)


def _tc3_body(ga_ref, gb_ref, w2_ref, b2_ref, w3_ref, b3_ref, w4_ref, b4_ref,
              wf1_ref, bf1_ref, wf2_ref, bf2_ref, wf3_ref, bf3_ref, out_ref):
    o = jnp.maximum(ga_ref[...] + gb_ref[...], 0.0)
    o = jnp.maximum(jnp.dot(o, w2_ref[...], preferred_element_type=jnp.float32) + b2_ref[...], 0.0)
    o = jnp.maximum(jnp.dot(o, w3_ref[...], preferred_element_type=jnp.float32) + b3_ref[...], 0.0)
    o = jnp.dot(o, w4_ref[...], preferred_element_type=jnp.float32) + b4_ref[...]
    o = jnp.maximum(jnp.dot(o, wf1_ref[...], preferred_element_type=jnp.float32) + bf1_ref[...], 0.0)
    o = jnp.maximum(jnp.dot(o, wf2_ref[...], preferred_element_type=jnp.float32) + bf2_ref[...], 0.0)
    out_ref[...] = jnp.dot(o, wf3_ref[...], preferred_element_type=jnp.float32) + bf3_ref[...]


def _full(shape):
    return pl.BlockSpec(shape, lambda i: tuple(0 for _ in shape))


def kernel(x, edge_index, y, Wl1, bl1, Wr1, Wl2, bl2, Wr2, Wo1, bo1, Wo2, bo2,
           Wo3, bo3, Wo4, bo4, Wf1, bf1, Wf2, bf2, Wf3, bf3):
    N = x.shape[0]
    E = edge_index.shape[1]
    C = Wf3.shape[1]
    src = edge_index[0]
    dst = edge_index[1]

    zeros16 = jnp.zeros((N, 16), jnp.float32)
    zeros32 = jnp.zeros((N, 32), jnp.float32)
    ones_rows = jnp.ones((CHUNK, 16), jnp.float32)

    xsum_parts, deg_parts = _sc_l1(src, dst, x, zeros16, ones_rows, N, E)
    xsum_parts = xsum_parts.reshape(NC, N, 16)
    deg_parts = deg_parts.reshape(NC, N, 16)

    BN = 5000
    h1s = pl.pallas_call(
        _tc1_body,
        grid=(N // BN,),
        in_specs=[
            pl.BlockSpec((NC, BN, 16), lambda i: (0, i, 0)),
            pl.BlockSpec((NC, BN, 16), lambda i: (0, i, 0)),
            pl.BlockSpec((BN, 16), lambda i: (i, 0)),
            _full((16, 64)),
            _full((1, 64)),
            _full((16, 64)),
        ],
        out_specs=pl.BlockSpec((NC, BN, 32), lambda i: (0, i, 0)),
        out_shape=jax.ShapeDtypeStruct((NC, N, 32), jnp.float32),
    )(xsum_parts, deg_parts, x, Wl1, bl1.reshape(1, 64), Wr1)

    agg2 = _sc_l2(src, dst, h1s.reshape(NC * N, 32), zeros32, N, E)
    agg2 = agg2.reshape(NC, N, 32)

    ab = pl.pallas_call(
        _tc2_body,
        grid=(N // BN,),
        in_specs=[
            pl.BlockSpec((NC, BN, 32), lambda i: (0, i, 0)),
            pl.BlockSpec((NC, BN, 16), lambda i: (0, i, 0)),
            pl.BlockSpec((NC, BN, 32), lambda i: (0, i, 0)),
            _full((64, 128)),
            _full((1, 128)),
            _full((64, 128)),
            _full((256, 128)),
            _full((1, 128)),
        ],
        out_specs=pl.BlockSpec((2, BN, 128), lambda i: (0, i, 0)),
        out_shape=jax.ShapeDtypeStruct((2, N, 128), jnp.float32),
    )(agg2, deg_parts, h1s, Wl2, bl2.reshape(1, 128), Wr2, Wo1,
      bo1.reshape(1, 128))

    ga, gb = _sc_pair_gather(src, dst, ab.reshape(2 * N, 128), N, E)

    BE = 3200
    r = pl.pallas_call(
        _tc3_body,
        grid=(E // BE,),
        in_specs=[
            pl.BlockSpec((BE, 128), lambda i: (i, 0)),
            pl.BlockSpec((BE, 128), lambda i: (i, 0)),
            _full((128, 64)), _full((1, 64)),
            _full((64, 32)), _full((1, 32)),
            _full((32, 16)), _full((1, 16)),
            _full((16, 8)), _full((1, 8)),
            _full((8, 4)), _full((1, 4)),
            _full((4, C)), _full((1, C)),
        ],
        out_specs=pl.BlockSpec((BE, C), lambda i: (i, 0)),
        out_shape=jax.ShapeDtypeStruct((E, C), jnp.float32),
    )(ga, gb, Wo2, bo2.reshape(1, 64), Wo3, bo3.reshape(1, 32),
      Wo4, bo4.reshape(1, 16), Wf1, bf1.reshape(1, 8), Wf2, bf2.reshape(1, 4),
      Wf3, bf3.reshape(1, C))

    return (r, y)


# speedup vs baseline: 7.3499x; 2.1672x over previous
import jax
import jax.numpy as jnp
from jax import lax
from jax.experimental import pallas as pl
from jax.experimental.pallas import tpu as pltpu
from jax.experimental.pallas import tpu_sc as plsc

NC = 2
NS = 16
CHUNK = 80
IDXG = 5


def _mesh():
    return plsc.VectorSubcoreMesh(
        core_axis_name="c", subcore_axis_name="s", num_cores=NC, num_subcores=NS
    )


def _row_split(s, n_rows, fn):
    nps = (-(-n_rows // NS) + 7) // 8 * 8
    last = n_rows - (NS - 1) * nps

    @pl.when(s < NS - 1)
    def _():
        fn(s * nps, nps)

    @pl.when(s == NS - 1)
    def _():
        fn((NS - 1) * nps, last)


def _grouped_loop(crow0, ngrp, src2d, dst2d, idx_s, idx_d, isem_s, isem_d,
                  emit_group):
    def prefetch(g, slot):
        crow = crow0 + g * IDXG
        pltpu.async_copy(src2d.at[pl.ds(crow, IDXG)], idx_s.at[slot], isem_s)
        pltpu.async_copy(dst2d.at[pl.ds(crow, IDXG)], idx_d.at[slot], isem_d)

    def group(g, gb):
        crow_g = crow0 + g * IDXG
        pltpu.make_async_copy(src2d.at[pl.ds(crow_g, IDXG)], idx_s.at[gb],
                              isem_s).wait()
        pltpu.make_async_copy(dst2d.at[pl.ds(crow_g, IDXG)], idx_d.at[gb],
                              isem_d).wait()

        @pl.when(g + 1 < ngrp)
        def _():
            prefetch(g + 1, 1 - gb)

        emit_group(g, gb)

    prefetch(0, 0)

    def pair_body(go, carry):
        group(2 * go, 0)
        group(2 * go + 1, 1)
        return carry

    lax.fori_loop(0, ngrp // 2, pair_body, 0)
    if ngrp % 2:
        group(ngrp - 1, (ngrp - 1) % 2)


def _sc_l1(src2d, dst2d, x, zeros16, ones_rows, N, E):
    CW = (E // CHUNK) // (NC * NS)
    NGRP = CW // IDXG

    def body(src_hbm, dst_hbm, x_hbm, z_hbm, ones_hbm,
             xsum_out, deg_out,
             acc_x, acc_d, idx_s, idx_d, rows, ones_v,
             isem_s, isem_d, g0, g1, g2, g3, g4, ssem_x, ssem_d):
        c = lax.axis_index("c")
        s = lax.axis_index("s")
        w = c * NS + s

        def zero(r0, nr):
            pltpu.sync_copy(z_hbm.at[pl.ds(r0, nr)], acc_x.at[pl.ds(r0, nr)])
            pltpu.sync_copy(z_hbm.at[pl.ds(r0, nr)], acc_d.at[pl.ds(r0, nr)])

        _row_split(s, N, zero)
        pltpu.sync_copy(ones_hbm, ones_v)
        plsc.subcore_barrier()

        gsems = [g0, g1, g2, g3, g4]

        def emit_group(g, gb):
            handles = []
            for k in range(IDXG):
                pltpu.async_copy(x_hbm.at[idx_s.at[gb, k]], rows.at[k],
                                 gsems[k])
            for k in range(IDXG):
                pltpu.make_async_copy(x_hbm.at[idx_s.at[gb, k]], rows.at[k],
                                      gsems[k]).wait()
                handles.append(pltpu.async_copy(
                    rows.at[k], acc_x.at[idx_d.at[gb, k]], ssem_x, add=True))
                handles.append(pltpu.async_copy(
                    ones_v, acc_d.at[idx_d.at[gb, k]], ssem_d, add=True))
            for h in handles:
                h.wait()

        _grouped_loop(w * CW, NGRP, src_hbm, dst_hbm, idx_s, idx_d,
                      isem_s, isem_d, emit_group)
        plsc.subcore_barrier()

        def writeout(r0, nr):
            pltpu.sync_copy(acc_x.at[pl.ds(r0, nr)],
                            xsum_out.at[pl.ds(c * N + r0, nr)])
            pltpu.sync_copy(acc_d.at[pl.ds(r0, nr)],
                            deg_out.at[pl.ds(c * N + r0, nr)])

        _row_split(s, N, writeout)

    f = pl.kernel(
        body,
        out_type=(
            jax.ShapeDtypeStruct((NC * N, 16), jnp.float32),
            jax.ShapeDtypeStruct((NC * N, 16), jnp.float32),
        ),
        mesh=_mesh(),
        compiler_params=pltpu.CompilerParams(use_tc_tiling_on_sc=False),
        scratch_types=[
            pltpu.VMEM_SHARED((N, 16), jnp.float32),
            pltpu.VMEM_SHARED((N, 16), jnp.float32),
            pltpu.VMEM((2, IDXG, CHUNK), jnp.int32),
            pltpu.VMEM((2, IDXG, CHUNK), jnp.int32),
            pltpu.VMEM((IDXG, CHUNK, 16), jnp.float32),
            pltpu.VMEM((CHUNK, 16), jnp.float32),
        ] + [pltpu.SemaphoreType.DMA] * 9,
    )
    return f(src2d, dst2d, x, zeros16, ones_rows)


def _sc_l2(src2d, dst2d, h1s, zeros32, N, E):
    CW = (E // CHUNK) // NS
    NGRP = CW // IDXG

    def body(src_hbm, dst_hbm, t_hbm, z_hbm, out_hbm,
             acc, idx_s, idx_d, rows,
             isem_s, isem_d, g0, g1, g2, g3, g4, ssem):
        c = lax.axis_index("c")
        s = lax.axis_index("s")
        _row_split(s, N, lambda r0, nr: pltpu.sync_copy(
            z_hbm.at[pl.ds(r0, nr)], acc.at[pl.ds(r0, nr)]))
        plsc.subcore_barrier()

        tab = t_hbm.at[pl.ds(c * N, N)]
        gsems = [g0, g1, g2, g3, g4]

        def emit_group(g, gb):
            handles = []
            for k in range(IDXG):
                pltpu.async_copy(tab.at[idx_s.at[gb, k]], rows.at[k], gsems[k])
            for k in range(IDXG):
                pltpu.make_async_copy(tab.at[idx_s.at[gb, k]], rows.at[k],
                                      gsems[k]).wait()
                handles.append(pltpu.async_copy(
                    rows.at[k], acc.at[idx_d.at[gb, k]], ssem, add=True))
            for h in handles:
                h.wait()

        _grouped_loop(s * CW, NGRP, src_hbm, dst_hbm, idx_s, idx_d,
                      isem_s, isem_d, emit_group)
        plsc.subcore_barrier()
        _row_split(s, N, lambda r0, nr: pltpu.sync_copy(
            acc.at[pl.ds(r0, nr)], out_hbm.at[pl.ds(c * N + r0, nr)]))

    f = pl.kernel(
        body,
        out_type=jax.ShapeDtypeStruct((NC * N, 32), jnp.float32),
        mesh=_mesh(),
        compiler_params=pltpu.CompilerParams(use_tc_tiling_on_sc=False),
        scratch_types=[
            pltpu.VMEM_SHARED((N, 32), jnp.float32),
            pltpu.VMEM((2, IDXG, CHUNK), jnp.int32),
            pltpu.VMEM((2, IDXG, CHUNK), jnp.int32),
            pltpu.VMEM((IDXG, CHUNK, 32), jnp.float32),
        ] + [pltpu.SemaphoreType.DMA] * 8,
    )
    return f(src2d, dst2d, h1s, zeros32)


def _sc_pair_gather(src2d, dst2d, tab, N, E):
    CW = (E // CHUNK) // (NC * NS)
    NGRP = CW // IDXG

    def body(src_hbm, dst_hbm, t_hbm, ga_out, gb_out,
             idx_s, idx_d, rows_a, rows_b,
             isem_s, isem_d, a0, a1, a2, a3, a4, b0, b1, b2, b3, b4,
             osem_a, osem_b):
        c = lax.axis_index("c")
        s = lax.axis_index("s")
        w = c * NS + s
        ta = t_hbm.at[pl.ds(0, N)]
        tb = t_hbm.at[pl.ds(N, N)]
        asems = [a0, a1, a2, a3, a4]
        bsems = [b0, b1, b2, b3, b4]

        def emit_group(g, gb):
            crow = w * CW + g * IDXG
            handles = []
            for k in range(IDXG):
                pltpu.async_copy(ta.at[idx_s.at[gb, k]], rows_a.at[k],
                                 asems[k])
                pltpu.async_copy(tb.at[idx_d.at[gb, k]], rows_b.at[k],
                                 bsems[k])
            for k in range(IDXG):
                base = (crow + k) * CHUNK
                pltpu.make_async_copy(ta.at[idx_s.at[gb, k]], rows_a.at[k],
                                      asems[k]).wait()
                handles.append(pltpu.async_copy(
                    rows_a.at[k], ga_out.at[pl.ds(base, CHUNK)], osem_a))
                pltpu.make_async_copy(tb.at[idx_d.at[gb, k]], rows_b.at[k],
                                      bsems[k]).wait()
                handles.append(pltpu.async_copy(
                    rows_b.at[k], gb_out.at[pl.ds(base, CHUNK)], osem_b))
            for h in handles:
                h.wait()

        _grouped_loop(w * CW, NGRP, src_hbm, dst_hbm, idx_s, idx_d,
                      isem_s, isem_d, emit_group)

    f = pl.kernel(
        body,
        out_type=(
            jax.ShapeDtypeStruct((E, 128), jnp.float32),
            jax.ShapeDtypeStruct((E, 128), jnp.float32),
        ),
        mesh=_mesh(),
        compiler_params=pltpu.CompilerParams(use_tc_tiling_on_sc=False),
        scratch_types=[
            pltpu.VMEM((2, IDXG, CHUNK), jnp.int32),
            pltpu.VMEM((2, IDXG, CHUNK), jnp.int32),
            pltpu.VMEM((IDXG, CHUNK, 128), jnp.float32),
            pltpu.VMEM((IDXG, CHUNK, 128), jnp.float32),
        ] + [pltpu.SemaphoreType.DMA] * 14,
    )
    return f(src2d, dst2d, tab)


def _tc1_body(px_ref, pd_ref, x_ref, wl_ref, bl_ref, wr_ref, out_ref):
    px = px_ref[...]
    pd = pd_ref[...]
    xs = px[0] + px[1]
    deg = pd[0, :, 0:1] + pd[1, :, 0:1]
    inv = 1.0 / jnp.maximum(deg, 1.0)
    agg = xs * inv
    h = jnp.dot(agg, wl_ref[...], preferred_element_type=jnp.float32)
    h = h + bl_ref[...]
    h = h + jnp.dot(x_ref[...], wr_ref[...], preferred_element_type=jnp.float32)
    h = jnp.maximum(h, 0.0)
    out_ref[0] = h[:, :32]
    out_ref[1] = h[:, 32:]


def _tc2_body(a2_ref, pd_ref, h1s_ref, wl_ref, bl_ref, wr_ref, wo1_ref,
              bo1_ref, out_ref):
    a2 = a2_ref[...]
    agg2 = jnp.concatenate([a2[0], a2[1]], axis=1)
    pd = pd_ref[...]
    deg = pd[0, :, 0:1] + pd[1, :, 0:1]
    inv = 1.0 / jnp.maximum(deg, 1.0)
    h1s = h1s_ref[...]
    h1 = jnp.concatenate([h1s[0], h1s[1]], axis=1)
    h2 = jnp.dot(agg2 * inv, wl_ref[...], preferred_element_type=jnp.float32)
    h2 = h2 + bl_ref[...]
    h2 = h2 + jnp.dot(h1, wr_ref[...], preferred_element_type=jnp.float32)
    h2 = jnp.maximum(h2, 0.0)
    wo1 = wo1_ref[...]
    out_ref[0] = jnp.dot(h2, wo1[:128], preferred_element_type=jnp.float32) + bo1_ref[...]
    out_ref[1] = jnp.dot(h2, wo1[128:], preferred_element_type=jnp.float32)


def _tc3_body(ga_ref, gb_ref, w2_ref, b2_ref, w3_ref, b3_ref, w4_ref, b4_ref,
              wf1_ref, bf1_ref, wf2_ref, bf2_ref, wf3_ref, bf3_ref, out_ref):
    o = jnp.maximum(ga_ref[...] + gb_ref[...], 0.0)
    o = jnp.maximum(jnp.dot(o, w2_ref[...], preferred_element_type=jnp.float32) + b2_ref[...], 0.0)
    o = jnp.maximum(jnp.dot(o, w3_ref[...], preferred_element_type=jnp.float32) + b3_ref[...], 0.0)
    o = jnp.dot(o, w4_ref[...], preferred_element_type=jnp.float32) + b4_ref[...]
    o = jnp.maximum(jnp.dot(o, wf1_ref[...], preferred_element_type=jnp.float32) + bf1_ref[...], 0.0)
    o = jnp.maximum(jnp.dot(o, wf2_ref[...], preferred_element_type=jnp.float32) + bf2_ref[...], 0.0)
    out_ref[...] = jnp.dot(o, wf3_ref[...], preferred_element_type=jnp.float32) + bf3_ref[...]


def _full(shape):
    return pl.BlockSpec(shape, lambda i: tuple(0 for _ in shape))


def kernel(x, edge_index, y, Wl1, bl1, Wr1, Wl2, bl2, Wr2, Wo1, bo1, Wo2, bo2,
           Wo3, bo3, Wo4, bo4, Wf1, bf1, Wf2, bf2, Wf3, bf3):
    N = x.shape[0]
    E = edge_index.shape[1]
    C = Wf3.shape[1]
    src2d = edge_index[0].reshape(E // CHUNK, CHUNK)
    dst2d = edge_index[1].reshape(E // CHUNK, CHUNK)

    zeros16 = jnp.zeros((N, 16), jnp.float32)
    zeros32 = jnp.zeros((N, 32), jnp.float32)
    ones_rows = jnp.ones((CHUNK, 16), jnp.float32)

    xsum_parts, deg_parts = _sc_l1(src2d, dst2d, x, zeros16, ones_rows, N, E)
    xsum_parts = xsum_parts.reshape(NC, N, 16)
    deg_parts = deg_parts.reshape(NC, N, 16)

    BN = 5000
    h1s = pl.pallas_call(
        _tc1_body,
        grid=(N // BN,),
        in_specs=[
            pl.BlockSpec((NC, BN, 16), lambda i: (0, i, 0)),
            pl.BlockSpec((NC, BN, 16), lambda i: (0, i, 0)),
            pl.BlockSpec((BN, 16), lambda i: (i, 0)),
            _full((16, 64)),
            _full((1, 64)),
            _full((16, 64)),
        ],
        out_specs=pl.BlockSpec((NC, BN, 32), lambda i: (0, i, 0)),
        out_shape=jax.ShapeDtypeStruct((NC, N, 32), jnp.float32),
    )(xsum_parts, deg_parts, x, Wl1, bl1.reshape(1, 64), Wr1)

    agg2 = _sc_l2(src2d, dst2d, h1s.reshape(NC * N, 32), zeros32, N, E)
    agg2 = agg2.reshape(NC, N, 32)

    ab = pl.pallas_call(
        _tc2_body,
        grid=(N // BN,),
        in_specs=[
            pl.BlockSpec((NC, BN, 32), lambda i: (0, i, 0)),
            pl.BlockSpec((NC, BN, 16), lambda i: (0, i, 0)),
            pl.BlockSpec((NC, BN, 32), lambda i: (0, i, 0)),
            _full((64, 128)),
            _full((1, 128)),
            _full((64, 128)),
            _full((256, 128)),
            _full((1, 128)),
        ],
        out_specs=pl.BlockSpec((2, BN, 128), lambda i: (0, i, 0)),
        out_shape=jax.ShapeDtypeStruct((2, N, 128), jnp.float32),
    )(agg2, deg_parts, h1s, Wl2, bl2.reshape(1, 128), Wr2, Wo1,
      bo1.reshape(1, 128))

    ga, gb = _sc_pair_gather(src2d, dst2d, ab.reshape(2 * N, 128), N, E)

    BE = 3200
    r = pl.pallas_call(
        _tc3_body,
        grid=(E // BE,),
        in_specs=[
            pl.BlockSpec((BE, 128), lambda i: (i, 0)),
            pl.BlockSpec((BE, 128), lambda i: (i, 0)),
            _full((128, 64)), _full((1, 64)),
            _full((64, 32)), _full((1, 32)),
            _full((32, 16)), _full((1, 16)),
            _full((16, 8)), _full((1, 8)),
            _full((8, 4)), _full((1, 4)),
            _full((4, C)), _full((1, C)),
        ],
        out_specs=pl.BlockSpec((BE, C), lambda i: (i, 0)),
        out_shape=jax.ShapeDtypeStruct((E, C), jnp.float32),
    )(ga, gb, Wo2, bo2.reshape(1, 64), Wo3, bo3.reshape(1, 32),
      Wo4, bo4.reshape(1, 16), Wf1, bf1.reshape(1, 8), Wf2, bf2.reshape(1, 4),
      Wf3, bf3.reshape(1, C))

    return (r, y)


# speedup vs baseline: 8.5182x; 1.1589x over previous
import jax
import jax.numpy as jnp
from jax import lax
from jax.experimental import pallas as pl
from jax.experimental.pallas import tpu as pltpu
from jax.experimental.pallas import tpu_sc as plsc

NC = 2
NS = 16
CHUNK = 80
IDXG = 5


def _mesh():
    return plsc.VectorSubcoreMesh(
        core_axis_name="c", subcore_axis_name="s", num_cores=NC, num_subcores=NS
    )


def _row_split(s, n_rows, fn):
    nps = (-(-n_rows // NS) + 7) // 8 * 8
    last = n_rows - (NS - 1) * nps

    @pl.when(s < NS - 1)
    def _():
        fn(s * nps, nps)

    @pl.when(s == NS - 1)
    def _():
        fn((NS - 1) * nps, last)


def _grouped_loop(crow0, ngrp, src2d, dst2d, idx_s, idx_d, isem_s, isem_d,
                  emit_group):
    def prefetch(g, slot):
        crow = crow0 + g * IDXG
        pltpu.async_copy(src2d.at[pl.ds(crow, IDXG)], idx_s.at[slot], isem_s)
        pltpu.async_copy(dst2d.at[pl.ds(crow, IDXG)], idx_d.at[slot], isem_d)

    def group(g, gb):
        crow_g = crow0 + g * IDXG
        pltpu.make_async_copy(src2d.at[pl.ds(crow_g, IDXG)], idx_s.at[gb],
                              isem_s).wait()
        pltpu.make_async_copy(dst2d.at[pl.ds(crow_g, IDXG)], idx_d.at[gb],
                              isem_d).wait()

        @pl.when(g + 1 < ngrp)
        def _():
            prefetch(g + 1, 1 - gb)

        emit_group(g, gb)

    prefetch(0, 0)

    def pair_body(go, carry):
        group(2 * go, 0)
        group(2 * go + 1, 1)
        return carry

    lax.fori_loop(0, ngrp // 2, pair_body, 0)
    if ngrp % 2:
        group(ngrp - 1, (ngrp - 1) % 2)


def _sc_l1(src2d, dst2d, x, zeros16, ones_rows, N, E):
    CW = (E // CHUNK) // (NC * NS)
    NGRP = CW // IDXG

    def body(src_hbm, dst_hbm, x_hbm, z_hbm, ones_hbm,
             xsum_out, deg_out,
             acc_x, acc_d, idx_s, idx_d, rows, ones_v,
             isem_s, isem_d, g0, g1, g2, g3, g4, ssem_x, ssem_d):
        c = lax.axis_index("c")
        s = lax.axis_index("s")
        w = c * NS + s

        def zero(r0, nr):
            pltpu.sync_copy(z_hbm.at[pl.ds(r0, nr)], acc_x.at[pl.ds(r0, nr)])
            pltpu.sync_copy(z_hbm.at[pl.ds(r0, nr)], acc_d.at[pl.ds(r0, nr)])

        _row_split(s, N, zero)
        pltpu.sync_copy(ones_hbm, ones_v)
        plsc.subcore_barrier()

        gsems = [g0, g1, g2, g3, g4]

        def emit_group(g, gb):
            handles = []
            for k in range(IDXG):
                pltpu.async_copy(x_hbm.at[idx_s.at[gb, k]], rows.at[k],
                                 gsems[k])
            for k in range(IDXG):
                pltpu.make_async_copy(x_hbm.at[idx_s.at[gb, k]], rows.at[k],
                                      gsems[k]).wait()
                handles.append(pltpu.async_copy(
                    rows.at[k], acc_x.at[idx_d.at[gb, k]], ssem_x, add=True))
                handles.append(pltpu.async_copy(
                    ones_v, acc_d.at[idx_d.at[gb, k]], ssem_d, add=True))
            for h in handles:
                h.wait()

        _grouped_loop(w * CW, NGRP, src_hbm, dst_hbm, idx_s, idx_d,
                      isem_s, isem_d, emit_group)
        plsc.subcore_barrier()

        def writeout(r0, nr):
            pltpu.sync_copy(acc_x.at[pl.ds(r0, nr)],
                            xsum_out.at[pl.ds(c * N + r0, nr)])
            pltpu.sync_copy(acc_d.at[pl.ds(r0, nr)],
                            deg_out.at[pl.ds(c * N + r0, nr)])

        _row_split(s, N, writeout)

    f = pl.kernel(
        body,
        out_type=(
            jax.ShapeDtypeStruct((NC * N, 16), jnp.float32),
            jax.ShapeDtypeStruct((NC * N, 16), jnp.float32),
        ),
        mesh=_mesh(),
        compiler_params=pltpu.CompilerParams(use_tc_tiling_on_sc=False),
        scratch_types=[
            pltpu.VMEM_SHARED((N, 16), jnp.float32),
            pltpu.VMEM_SHARED((N, 16), jnp.float32),
            pltpu.VMEM((2, IDXG, CHUNK), jnp.int32),
            pltpu.VMEM((2, IDXG, CHUNK), jnp.int32),
            pltpu.VMEM((IDXG, CHUNK, 16), jnp.float32),
            pltpu.VMEM((CHUNK, 16), jnp.float32),
        ] + [pltpu.SemaphoreType.DMA] * 9,
    )
    return f(src2d, dst2d, x, zeros16, ones_rows)


def _sc_l2(src2d, dst2d, h1s, zeros32, N, E):
    CW = (E // CHUNK) // NS
    NGRP = CW // IDXG

    def body(src_hbm, dst_hbm, t_hbm, z_hbm, out_hbm,
             acc, idx_s, idx_d, rows,
             isem_s, isem_d, g0, g1, g2, g3, g4, ssem):
        c = lax.axis_index("c")
        s = lax.axis_index("s")
        _row_split(s, N, lambda r0, nr: pltpu.sync_copy(
            z_hbm.at[pl.ds(r0, nr)], acc.at[pl.ds(r0, nr)]))
        plsc.subcore_barrier()

        tab = t_hbm.at[pl.ds(c * N, N)]
        gsems = [g0, g1, g2, g3, g4]

        def emit_group(g, gb):
            handles = []
            for k in range(IDXG):
                pltpu.async_copy(tab.at[idx_s.at[gb, k]], rows.at[k], gsems[k])
            for k in range(IDXG):
                pltpu.make_async_copy(tab.at[idx_s.at[gb, k]], rows.at[k],
                                      gsems[k]).wait()
                handles.append(pltpu.async_copy(
                    rows.at[k], acc.at[idx_d.at[gb, k]], ssem, add=True))
            for h in handles:
                h.wait()

        _grouped_loop(s * CW, NGRP, src_hbm, dst_hbm, idx_s, idx_d,
                      isem_s, isem_d, emit_group)
        plsc.subcore_barrier()
        _row_split(s, N, lambda r0, nr: pltpu.sync_copy(
            acc.at[pl.ds(r0, nr)], out_hbm.at[pl.ds(c * N + r0, nr)]))

    f = pl.kernel(
        body,
        out_type=jax.ShapeDtypeStruct((NC * N, 32), jnp.float32),
        mesh=_mesh(),
        compiler_params=pltpu.CompilerParams(use_tc_tiling_on_sc=False),
        scratch_types=[
            pltpu.VMEM_SHARED((N, 32), jnp.float32),
            pltpu.VMEM((2, IDXG, CHUNK), jnp.int32),
            pltpu.VMEM((2, IDXG, CHUNK), jnp.int32),
            pltpu.VMEM((IDXG, CHUNK, 32), jnp.float32),
        ] + [pltpu.SemaphoreType.DMA] * 8,
    )
    return f(src2d, dst2d, h1s, zeros32)


def _sc_pair_gather(src2d, dst2d, tab, N, crow0, ncrow):
    CW = ncrow // (NC * NS)
    NGRP = CW // IDXG
    ES = ncrow * CHUNK

    def body(src_hbm, dst_hbm, t_hbm, ga_out, gb_out,
             idx_s, idx_d, rows_a, rows_b,
             isem_s, isem_d, a0, a1, a2, a3, a4, b0, b1, b2, b3, b4,
             osem_a, osem_b):
        c = lax.axis_index("c")
        s = lax.axis_index("s")
        w = c * NS + s
        ta = t_hbm.at[pl.ds(0, N)]
        tb = t_hbm.at[pl.ds(N, N)]
        asems = [a0, a1, a2, a3, a4]
        bsems = [b0, b1, b2, b3, b4]

        def emit_group(g, gb):
            crow = w * CW + g * IDXG
            handles = []

            for k in range(IDXG):
                pltpu.async_copy(ta.at[idx_s.at[gb, k]], rows_a.at[k],
                                 asems[k])
                pltpu.async_copy(tb.at[idx_d.at[gb, k]], rows_b.at[k],
                                 bsems[k])
            for k in range(IDXG):
                base = (crow + k) * CHUNK
                pltpu.make_async_copy(ta.at[idx_s.at[gb, k]], rows_a.at[k],
                                      asems[k]).wait()
                handles.append(pltpu.async_copy(
                    rows_a.at[k], ga_out.at[pl.ds(base, CHUNK)], osem_a))
                pltpu.make_async_copy(tb.at[idx_d.at[gb, k]], rows_b.at[k],
                                      bsems[k]).wait()
                handles.append(pltpu.async_copy(
                    rows_b.at[k], gb_out.at[pl.ds(base, CHUNK)], osem_b))
            for h in handles:
                h.wait()

        _grouped_loop(crow0 + w * CW, NGRP, src_hbm, dst_hbm, idx_s, idx_d,
                      isem_s, isem_d, emit_group)

    f = pl.kernel(
        body,
        out_type=(
            jax.ShapeDtypeStruct((ES, 128), jnp.float32),
            jax.ShapeDtypeStruct((ES, 128), jnp.float32),
        ),
        mesh=_mesh(),
        compiler_params=pltpu.CompilerParams(use_tc_tiling_on_sc=False),
        scratch_types=[
            pltpu.VMEM((2, IDXG, CHUNK), jnp.int32),
            pltpu.VMEM((2, IDXG, CHUNK), jnp.int32),
            pltpu.VMEM((IDXG, CHUNK, 128), jnp.float32),
            pltpu.VMEM((IDXG, CHUNK, 128), jnp.float32),
        ] + [pltpu.SemaphoreType.DMA] * 14,
    )
    return f(src2d, dst2d, tab)


def _tc1_body(px_ref, pd_ref, x_ref, wl_ref, bl_ref, wr_ref, out_ref):
    px = px_ref[...]
    pd = pd_ref[...]
    xs = px[0] + px[1]
    deg = pd[0, :, 0:1] + pd[1, :, 0:1]
    inv = 1.0 / jnp.maximum(deg, 1.0)
    agg = xs * inv
    h = jnp.dot(agg, wl_ref[...], preferred_element_type=jnp.float32)
    h = h + bl_ref[...]
    h = h + jnp.dot(x_ref[...], wr_ref[...], preferred_element_type=jnp.float32)
    h = jnp.maximum(h, 0.0)
    out_ref[0] = h[:, :32]
    out_ref[1] = h[:, 32:]


def _tc2_body(a2_ref, pd_ref, h1s_ref, wl_ref, bl_ref, wr_ref, wo1_ref,
              bo1_ref, out_ref):
    a2 = a2_ref[...]
    agg2 = jnp.concatenate([a2[0], a2[1]], axis=1)
    pd = pd_ref[...]
    deg = pd[0, :, 0:1] + pd[1, :, 0:1]
    inv = 1.0 / jnp.maximum(deg, 1.0)
    h1s = h1s_ref[...]
    h1 = jnp.concatenate([h1s[0], h1s[1]], axis=1)
    h2 = jnp.dot(agg2 * inv, wl_ref[...], preferred_element_type=jnp.float32)
    h2 = h2 + bl_ref[...]
    h2 = h2 + jnp.dot(h1, wr_ref[...], preferred_element_type=jnp.float32)
    h2 = jnp.maximum(h2, 0.0)
    wo1 = wo1_ref[...]
    out_ref[0] = jnp.dot(h2, wo1[:128], preferred_element_type=jnp.float32) + bo1_ref[...]
    out_ref[1] = jnp.dot(h2, wo1[128:], preferred_element_type=jnp.float32)


def _tc3_body(ga_ref, gb_ref, w2_ref, b2_ref, w3_ref, b3_ref, w4_ref, b4_ref,
              wf1_ref, bf1_ref, wf2_ref, bf2_ref, wf3_ref, bf3_ref, out_ref):
    o = jnp.maximum(ga_ref[...] + gb_ref[...], 0.0)
    o = jnp.maximum(jnp.dot(o, w2_ref[...], preferred_element_type=jnp.float32) + b2_ref[...], 0.0)
    o = jnp.maximum(jnp.dot(o, w3_ref[...], preferred_element_type=jnp.float32) + b3_ref[...], 0.0)
    o = jnp.dot(o, w4_ref[...], preferred_element_type=jnp.float32) + b4_ref[...]
    o = jnp.maximum(jnp.dot(o, wf1_ref[...], preferred_element_type=jnp.float32) + bf1_ref[...], 0.0)
    o = jnp.maximum(jnp.dot(o, wf2_ref[...], preferred_element_type=jnp.float32) + bf2_ref[...], 0.0)
    out_ref[...] = jnp.dot(o, wf3_ref[...], preferred_element_type=jnp.float32) + bf3_ref[...]


def _full(shape):
    return pl.BlockSpec(shape, lambda i: tuple(0 for _ in shape))


def kernel(x, edge_index, y, Wl1, bl1, Wr1, Wl2, bl2, Wr2, Wo1, bo1, Wo2, bo2,
           Wo3, bo3, Wo4, bo4, Wf1, bf1, Wf2, bf2, Wf3, bf3):
    N = x.shape[0]
    E = edge_index.shape[1]
    C = Wf3.shape[1]
    src2d = edge_index[0].reshape(E // CHUNK, CHUNK)
    dst2d = edge_index[1].reshape(E // CHUNK, CHUNK)

    zeros16 = jnp.zeros((N, 16), jnp.float32)
    zeros32 = jnp.zeros((N, 32), jnp.float32)
    ones_rows = jnp.ones((CHUNK, 16), jnp.float32)

    xsum_parts, deg_parts = _sc_l1(src2d, dst2d, x, zeros16, ones_rows, N, E)
    xsum_parts = xsum_parts.reshape(NC, N, 16)
    deg_parts = deg_parts.reshape(NC, N, 16)

    BN = 5000
    h1s = pl.pallas_call(
        _tc1_body,
        grid=(N // BN,),
        in_specs=[
            pl.BlockSpec((NC, BN, 16), lambda i: (0, i, 0)),
            pl.BlockSpec((NC, BN, 16), lambda i: (0, i, 0)),
            pl.BlockSpec((BN, 16), lambda i: (i, 0)),
            _full((16, 64)),
            _full((1, 64)),
            _full((16, 64)),
        ],
        out_specs=pl.BlockSpec((NC, BN, 32), lambda i: (0, i, 0)),
        out_shape=jax.ShapeDtypeStruct((NC, N, 32), jnp.float32),
    )(xsum_parts, deg_parts, x, Wl1, bl1.reshape(1, 64), Wr1)

    agg2 = _sc_l2(src2d, dst2d, h1s.reshape(NC * N, 32), zeros32, N, E)
    agg2 = agg2.reshape(NC, N, 32)

    ab = pl.pallas_call(
        _tc2_body,
        grid=(N // BN,),
        in_specs=[
            pl.BlockSpec((NC, BN, 32), lambda i: (0, i, 0)),
            pl.BlockSpec((NC, BN, 16), lambda i: (0, i, 0)),
            pl.BlockSpec((NC, BN, 32), lambda i: (0, i, 0)),
            _full((64, 128)),
            _full((1, 128)),
            _full((64, 128)),
            _full((256, 128)),
            _full((1, 128)),
        ],
        out_specs=pl.BlockSpec((2, BN, 128), lambda i: (0, i, 0)),
        out_shape=jax.ShapeDtypeStruct((2, N, 128), jnp.float32),
    )(agg2, deg_parts, h1s, Wl2, bl2.reshape(1, 128), Wr2, Wo1,
      bo1.reshape(1, 128))

    NSLICE = 5
    BE = 6400
    tab = ab.reshape(2 * N, 128)
    ncrow = (E // CHUNK) // NSLICE
    es = ncrow * CHUNK
    r_slices = []
    for sl in range(NSLICE):
        ga, gb = _sc_pair_gather(src2d, dst2d, tab, N, sl * ncrow, ncrow)
        r_slices.append(pl.pallas_call(
            _tc3_body,
            grid=(es // BE,),
            in_specs=[
                pl.BlockSpec((BE, 128), lambda i: (i, 0)),
                pl.BlockSpec((BE, 128), lambda i: (i, 0)),
                _full((128, 64)), _full((1, 64)),
                _full((64, 32)), _full((1, 32)),
                _full((32, 16)), _full((1, 16)),
                _full((16, 8)), _full((1, 8)),
                _full((8, 4)), _full((1, 4)),
                _full((4, C)), _full((1, C)),
            ],
            out_specs=pl.BlockSpec((BE, C), lambda i: (i, 0)),
            out_shape=jax.ShapeDtypeStruct((es, C), jnp.float32),
        )(ga, gb, Wo2, bo2.reshape(1, 64), Wo3, bo3.reshape(1, 32),
          Wo4, bo4.reshape(1, 16), Wf1, bf1.reshape(1, 8), Wf2,
          bf2.reshape(1, 4), Wf3, bf3.reshape(1, C)))

    r = jnp.concatenate(r_slices, axis=0)
    return (r, y)
